# Initial kernel scaffold; baseline (speedup 1.0000x reference)
import functools

import jax
import jax.numpy as jnp
from jax import lax
from jax.experimental import pallas as pl
from jax.experimental.pallas import tpu as pltpu
from jax.experimental.pallas import tpu_sc as plsc

N = 10000
E = 320000
D = 128
P = 16384

NC = 2
NS = 16
NW = NC * NS
L = 16

EW = E // NW
CB = 125
CH = EW // CB
ACC_ROWS = 10240
DUMMY = N

_mesh = plsc.VectorSubcoreMesh(
    core_axis_name="c", subcore_axis_name="s", num_cores=NC, num_subcores=NS)


def _seg_body(table_h, sidx_h, didx_h, part_h, cnt_h,
              sidx_v, didx_v, rows_v, hist_v, acc_s, gsem, ssem,
              *, with_counts):
    c = lax.axis_index("c")
    s = lax.axis_index("s")
    wid = s * NC + c
    zeros16 = jnp.zeros((L,), jnp.float32)
    ones16 = jnp.ones((L,), jnp.float32)

    pltpu.sync_copy(sidx_h.at[wid], sidx_v)
    pltpu.sync_copy(didx_h.at[wid], didx_v)

    def zrow(i, _):
        for k in range(8):
            rows_v[i, pl.ds(k * L, L)] = zeros16
        return 0
    lax.fori_loop(0, 128, zrow, 0)
    if with_counts:
        def zh(i, _):
            hist_v[pl.ds(i * L, L)] = zeros16
            return 0
        lax.fori_loop(0, (N + L) // L, zh, 0)
    for j in range(5):
        pltpu.sync_copy(rows_v, acc_s.at[pl.ds(s * 640 + j * 128, 128)])
    plsc.subcore_barrier()

    def chunk(j, _):
        pltpu.async_copy(table_h.at[sidx_v.at[j]], rows_v, gsem).wait()
        pltpu.async_copy(rows_v, acc_s.at[didx_v.at[j]], ssem, add=True).wait()
        if with_counts:
            for k in range(8):
                di = didx_v[j, pl.ds(k * L, L)]
                plsc.addupdate_scatter(hist_v, [di], ones16)
        return 0
    lax.fori_loop(0, CH, chunk, 0)
    plsc.subcore_barrier()

    rpt = N // NS
    pltpu.sync_copy(acc_s.at[pl.ds(s * rpt, rpt)], part_h.at[c, pl.ds(s * rpt, rpt)])
    if with_counts:
        pltpu.sync_copy(hist_v.at[pl.ds(0, N)], cnt_h.at[wid])


def _make_segsum(with_counts):
    out_type = (jax.ShapeDtypeStruct((NC, N, D), jnp.float32),
                jax.ShapeDtypeStruct((NW, N), jnp.float32))
    scratch = [
        pltpu.VMEM((CH, 128), jnp.int32),
        pltpu.VMEM((CH, 128), jnp.int32),
        pltpu.VMEM((128, D), jnp.float32),
        pltpu.VMEM((N + L,), jnp.float32),
        pltpu.VMEM_SHARED((ACC_ROWS, D), jnp.float32),
        pltpu.SemaphoreType.DMA,
        pltpu.SemaphoreType.DMA,
    ]
    return pl.kernel(
        functools.partial(_seg_body, with_counts=with_counts),
        out_type=out_type, mesh=_mesh, scratch_types=scratch,
        name=f"sc_segsum_counts{int(with_counts)}")


_segsum_counts = _make_segsum(True)
_segsum_plain = _make_segsum(False)


def _pair_body(h_h, pidx_h, out_h, pidx_v, rows_v, sem):
    c = lax.axis_index("c")
    s = lax.axis_index("s")
    wid = s * NC + c
    pltpu.sync_copy(pidx_h.at[wid], pidx_v)
    base = wid * (P // NW)
    for j in range(8):
        pltpu.async_copy(h_h.at[pidx_v.at[j]], rows_v, sem).wait()
        pltpu.sync_copy(
            rows_v, out_h.at[j // 4, pl.ds(base + (j % 4) * 128, 128)])


_pair_gather = pl.kernel(
    _pair_body,
    out_type=jax.ShapeDtypeStruct((2, P, D), jnp.float32),
    mesh=_mesh,
    scratch_types=[
        pltpu.VMEM((8, 128), jnp.int32),
        pltpu.VMEM((128, D), jnp.float32),
        pltpu.SemaphoreType.DMA,
    ],
    name="sc_pair_gather")


def _layer_tc_body(x_ref, p_ref, cnt_ref, wr_ref, wn_ref, b_ref, o_ref, *, relu):
    cnt = jnp.sum(cnt_ref[...], axis=0)
    inv = 1.0 / jnp.maximum(cnt, 1.0)
    agg = (p_ref[0] + p_ref[1]) * inv[:, None]
    out = (jnp.dot(x_ref[...], wr_ref[...], preferred_element_type=jnp.float32)
           + jnp.dot(agg, wn_ref[...], preferred_element_type=jnp.float32)
           + b_ref[...])
    o_ref[...] = jnp.maximum(out, 0.0) if relu else out


def _layer_tc(x, parts, cnts, wr, wn, b, relu):
    R = 2000
    grid = (N // R,)
    return pl.pallas_call(
        functools.partial(_layer_tc_body, relu=relu),
        grid=grid,
        in_specs=[
            pl.BlockSpec((R, D), lambda i: (i, 0)),
            pl.BlockSpec((NC, R, D), lambda i: (0, i, 0)),
            pl.BlockSpec((NW, R), lambda i: (0, i)),
            pl.BlockSpec((D, D), lambda i: (0, 0)),
            pl.BlockSpec((D, D), lambda i: (0, 0)),
            pl.BlockSpec((1, D), lambda i: (0, 0)),
        ],
        out_specs=pl.BlockSpec((R, D), lambda i: (i, 0)),
        out_shape=jax.ShapeDtypeStruct((N, D), jnp.float32),
    )(x, parts, cnts, wr, wn, b.reshape(1, D))


def _pred_body(hs_ref, hd_ref, w0_ref, b0_ref, w1_ref, b1_ref, w2_ref, b2_ref, o_ref):
    z = hs_ref[...] * hd_ref[...]
    z = jnp.maximum(jnp.dot(z, w0_ref[...], preferred_element_type=jnp.float32)
                    + b0_ref[...], 0.0)
    z = jnp.maximum(jnp.dot(z, w1_ref[...], preferred_element_type=jnp.float32)
                    + b1_ref[...], 0.0)
    logit = jnp.sum(z * w2_ref[...], axis=1, keepdims=True) + b2_ref[0, 0]
    o_ref[...] = 1.0 / (1.0 + jnp.exp(-logit))


def _predictor_tc(hs, hd, p0w, p0b, p1w, p1b, p2w, p2b):
    R = 2048
    grid = (P // R,)
    return pl.pallas_call(
        _pred_body,
        grid=grid,
        in_specs=[
            pl.BlockSpec((R, D), lambda i: (i, 0)),
            pl.BlockSpec((R, D), lambda i: (i, 0)),
            pl.BlockSpec((D, D), lambda i: (0, 0)),
            pl.BlockSpec((1, D), lambda i: (0, 0)),
            pl.BlockSpec((D, D), lambda i: (0, 0)),
            pl.BlockSpec((1, D), lambda i: (0, 0)),
            pl.BlockSpec((1, D), lambda i: (0, 0)),
            pl.BlockSpec((1, 1), lambda i: (0, 0)),
        ],
        out_specs=pl.BlockSpec((R, 1), lambda i: (i, 0)),
        out_shape=jax.ShapeDtypeStruct((P, 1), jnp.float32),
    )(hs, hd, p0w, p0b.reshape(1, D), p1w, p1b.reshape(1, D),
      p2w.reshape(1, D), p2b.reshape(1, 1))


def kernel(x, edge_index, pairs, W_root0, W_neigh0, b0, W_root1, W_neigh1, b1,
           P0_w, P0_b, P1_w, P1_b, P2_w, P2_b):
    src = edge_index[0].reshape(NW, CH, CB)
    dst = edge_index[1].reshape(NW, CH, CB)
    src_p = jnp.pad(src, ((0, 0), (0, 0), (0, 128 - CB)))
    dst_p = jnp.pad(dst, ((0, 0), (0, 0), (0, 128 - CB)), constant_values=DUMMY)

    parts0, cnts = _segsum_counts(x, src_p, dst_p)
    h = _layer_tc(x, parts0, cnts, W_root0, W_neigh0, b0, relu=True)

    parts1, _ = _segsum_plain(h, src_p, dst_p)
    h1 = _layer_tc(h, parts1, cnts, W_root1, W_neigh1, b1, relu=False)

    pidx = pairs.reshape(2, NW, 4, 128).transpose(1, 0, 2, 3).reshape(NW, 8, 128)
    hp = _pair_gather(h1, pidx)

    return _predictor_tc(hp[0], hp[1], P0_w, P0_b, P1_w, P1_b, P2_w, P2_b)


# speedup vs baseline: 4.4932x; 4.4932x over previous
import functools

import jax
import jax.numpy as jnp
from jax import lax
from jax.experimental import pallas as pl
from jax.experimental.pallas import tpu as pltpu
from jax.experimental.pallas import tpu_sc as plsc

N = 10000
E = 320000
D = 128
P = 16384

NC = 2
NS = 16
NW = NC * NS
L = 16

EW = E // NW
CB = 125
CH = EW // CB
ACC_ROWS = 10240
HR = ACC_ROWS // 128
DUMMY = N

_mesh = plsc.VectorSubcoreMesh(
    core_axis_name="c", subcore_axis_name="s", num_cores=NC, num_subcores=NS)


def _seg_body(table_h, sidx_h, didx_h, part_h, cnt_h,
              sidx_v, didx_v, rows_v, hist_v, acc_s, gsem, ssem,
              *, with_counts):
    c = lax.axis_index("c")
    s = lax.axis_index("s")
    wid = s * NC + c
    zeros16 = jnp.zeros((L,), jnp.float32)
    ones16 = jnp.ones((L,), jnp.float32)

    pltpu.sync_copy(sidx_h.at[wid], sidx_v)
    pltpu.sync_copy(didx_h.at[wid], didx_v)

    def zrow(i, _):
        for k in range(8):
            rows_v[i, pl.ds(k * L, L)] = zeros16
        return 0
    lax.fori_loop(0, 128, zrow, 0)
    if with_counts:
        def zh(i, _):
            hist_v[pl.ds(i * L, L)] = zeros16
            return 0
        lax.fori_loop(0, ACC_ROWS // L, zh, 0)
    for j in range(5):
        pltpu.sync_copy(rows_v, acc_s.at[pl.ds(s * 640 + j * 128, 128)])
    plsc.subcore_barrier()

    def chunk(j, _):
        pltpu.async_copy(table_h.at[sidx_v.at[j]], rows_v, gsem).wait()
        pltpu.async_copy(rows_v, acc_s.at[didx_v.at[j]], ssem, add=True).wait()
        if with_counts:
            for k in range(8):
                di = didx_v[j, pl.ds(k * L, L)]
                plsc.addupdate_scatter(hist_v, [di], ones16)
        return 0
    lax.fori_loop(0, CH, chunk, 0)
    plsc.subcore_barrier()

    pltpu.sync_copy(acc_s.at[pl.ds(s * 640, 640)], part_h.at[c, pl.ds(s * 640, 640)])
    if with_counts:
        pltpu.sync_copy(hist_v, cnt_h.at[pl.ds(wid * ACC_ROWS, ACC_ROWS)])


def _make_segsum(with_counts):
    out_type = (jax.ShapeDtypeStruct((NC, ACC_ROWS, D), jnp.float32),
                jax.ShapeDtypeStruct((NW * ACC_ROWS,), jnp.float32))
    scratch = [
        pltpu.VMEM((CH, 128), jnp.int32),
        pltpu.VMEM((CH, 128), jnp.int32),
        pltpu.VMEM((128, D), jnp.float32),
        pltpu.VMEM((ACC_ROWS,), jnp.float32),
        pltpu.VMEM_SHARED((ACC_ROWS, D), jnp.float32),
        pltpu.SemaphoreType.DMA,
        pltpu.SemaphoreType.DMA,
    ]
    return pl.kernel(
        functools.partial(_seg_body, with_counts=with_counts),
        out_type=out_type, mesh=_mesh, scratch_types=scratch,
        compiler_params=pltpu.CompilerParams(needs_layout_passes=False),
        name=f"sc_segsum_counts{int(with_counts)}")


_segsum_counts = _make_segsum(True)
_segsum_plain = _make_segsum(False)


def _pair_body(h_h, pidx_h, out_h, pidx_v, rows_v, sem):
    c = lax.axis_index("c")
    s = lax.axis_index("s")
    wid = s * NC + c
    pltpu.sync_copy(pidx_h.at[wid], pidx_v)
    base = wid * (P // NW)
    for j in range(8):
        pltpu.async_copy(h_h.at[pidx_v.at[j]], rows_v, sem).wait()
        pltpu.sync_copy(
            rows_v, out_h.at[j // 4, pl.ds(base + (j % 4) * 128, 128)])


_pair_gather = pl.kernel(
    _pair_body,
    out_type=jax.ShapeDtypeStruct((2, P, D), jnp.float32),
    mesh=_mesh,
    scratch_types=[
        pltpu.VMEM((8, 128), jnp.int32),
        pltpu.VMEM((128, D), jnp.float32),
        pltpu.SemaphoreType.DMA,
    ],
    compiler_params=pltpu.CompilerParams(needs_layout_passes=False),
    name="sc_pair_gather")


def _cnt_body(cnt_ref, inv_ref):
    tot = jnp.sum(cnt_ref[...], axis=0)
    inv_ref[...] = 1.0 / jnp.maximum(tot, 1.0)


def _cnt_reduce(cnts):
    return pl.pallas_call(
        _cnt_body,
        out_shape=jax.ShapeDtypeStruct((HR, 128), jnp.float32),
    )(cnts)


def _layer_tc_body(x_ref, p_ref, inv_ref, wr_ref, wn_ref, b_ref, o_ref, *, relu):
    agg = (p_ref[0] + p_ref[1]) * inv_ref[...]
    out = (jnp.dot(x_ref[...], wr_ref[...], preferred_element_type=jnp.float32)
           + jnp.dot(agg, wn_ref[...], preferred_element_type=jnp.float32)
           + b_ref[...])
    o_ref[...] = jnp.maximum(out, 0.0) if relu else out


def _layer_tc(x, parts, inv, wr, wn, b, relu):
    R = 2000
    grid = (N // R,)
    return pl.pallas_call(
        functools.partial(_layer_tc_body, relu=relu),
        grid=grid,
        in_specs=[
            pl.BlockSpec((R, D), lambda i: (i, 0)),
            pl.BlockSpec((NC, R, D), lambda i: (0, i, 0)),
            pl.BlockSpec((R, 1), lambda i: (i, 0)),
            pl.BlockSpec((D, D), lambda i: (0, 0)),
            pl.BlockSpec((D, D), lambda i: (0, 0)),
            pl.BlockSpec((1, D), lambda i: (0, 0)),
        ],
        out_specs=pl.BlockSpec((R, D), lambda i: (i, 0)),
        out_shape=jax.ShapeDtypeStruct((N, D), jnp.float32),
    )(x, parts, inv, wr, wn, b.reshape(1, D))


def _pred_body(hs_ref, hd_ref, w0_ref, b0_ref, w1_ref, b1_ref, w2_ref, b2_ref, o_ref):
    z = hs_ref[...] * hd_ref[...]
    z = jnp.maximum(jnp.dot(z, w0_ref[...], preferred_element_type=jnp.float32)
                    + b0_ref[...], 0.0)
    z = jnp.maximum(jnp.dot(z, w1_ref[...], preferred_element_type=jnp.float32)
                    + b1_ref[...], 0.0)
    logit = jnp.sum(z * w2_ref[...], axis=1, keepdims=True) + b2_ref[0, 0]
    o_ref[...] = 1.0 / (1.0 + jnp.exp(-logit))


def _predictor_tc(hs, hd, p0w, p0b, p1w, p1b, p2w, p2b):
    R = 2048
    grid = (P // R,)
    return pl.pallas_call(
        _pred_body,
        grid=grid,
        in_specs=[
            pl.BlockSpec((R, D), lambda i: (i, 0)),
            pl.BlockSpec((R, D), lambda i: (i, 0)),
            pl.BlockSpec((D, D), lambda i: (0, 0)),
            pl.BlockSpec((1, D), lambda i: (0, 0)),
            pl.BlockSpec((D, D), lambda i: (0, 0)),
            pl.BlockSpec((1, D), lambda i: (0, 0)),
            pl.BlockSpec((1, D), lambda i: (0, 0)),
            pl.BlockSpec((1, 1), lambda i: (0, 0)),
        ],
        out_specs=pl.BlockSpec((R, 1), lambda i: (i, 0)),
        out_shape=jax.ShapeDtypeStruct((P, 1), jnp.float32),
    )(hs, hd, p0w, p0b.reshape(1, D), p1w, p1b.reshape(1, D),
      p2w.reshape(1, D), p2b.reshape(1, 1))


def kernel(x, edge_index, pairs, W_root0, W_neigh0, b0, W_root1, W_neigh1, b1,
           P0_w, P0_b, P1_w, P1_b, P2_w, P2_b):
    src = edge_index[0].reshape(NW, CH, CB)
    dst = edge_index[1].reshape(NW, CH, CB)
    src_p = jnp.pad(src, ((0, 0), (0, 0), (0, 128 - CB)))
    dst_p = jnp.pad(dst, ((0, 0), (0, 0), (0, 128 - CB)), constant_values=DUMMY)

    parts0, cnts = _segsum_counts(x, src_p, dst_p)
    inv = _cnt_reduce(cnts.reshape(NW, HR, 128)).reshape(ACC_ROWS, 1)
    h = _layer_tc(x, parts0, inv, W_root0, W_neigh0, b0, relu=True)

    parts1, _ = _segsum_plain(h, src_p, dst_p)
    h1 = _layer_tc(h, parts1, inv, W_root1, W_neigh1, b1, relu=False)

    pidx = pairs.reshape(2, NW, 4, 128).transpose(1, 0, 2, 3).reshape(NW, 8, 128)
    hp = _pair_gather(h1, pidx)

    return _predictor_tc(hp[0], hp[1], P0_w, P0_b, P1_w, P1_b, P2_w, P2_b)
